# baseline (speedup 1.0000x reference)
import functools

import jax
import jax.numpy as jnp
from jax import lax
from jax.experimental import pallas as pl
from jax.experimental.pallas import tpu as pltpu
from jax.experimental.pallas import tpu_sc as plsc

N = 10000
D = 128
LAYERS = 2
NC = 2
NS = 16
CHUNK = 64
NP_ = 10240
ROWS_PER_TILE = NP_ // NS
CPT = 160
NROWBUF = 4
NIDX = 8


def _sc_agg(ego, col, row, w):
    mesh = plsc.VectorSubcoreMesh(core_axis_name="c", subcore_axis_name="s")

    @functools.partial(
        pl.kernel,
        mesh=mesh,
        out_type=jax.ShapeDtypeStruct((NC * NP_, D), jnp.float32),
        scratch_types=(
            [pltpu.VMEM_SHARED((NP_, D), jnp.float32)]
            + [pltpu.VMEM((CHUNK,), jnp.int32)] * NIDX
            + [pltpu.VMEM((CHUNK,), jnp.int32)] * NIDX
            + [pltpu.VMEM((CHUNK,), jnp.float32)] * NIDX
            + [pltpu.VMEM((CHUNK, D), jnp.float32)] * NROWBUF
            + [pltpu.SemaphoreType.DMA] * NROWBUF
            + [pltpu.SemaphoreType.DMA] * NROWBUF
            + [pltpu.SemaphoreType.DMA] * NIDX
        ),
    )
    def k(ego_hbm, col_hbm, row_hbm, w_hbm, out_hbm, agg_sh, *scr):
        colr = scr[0:NIDX]
        rowr = scr[NIDX:2 * NIDX]
        wr = scr[2 * NIDX:3 * NIDX]
        rows = scr[3 * NIDX:3 * NIDX + NROWBUF]
        sem_g = scr[3 * NIDX + NROWBUF:3 * NIDX + 2 * NROWBUF]
        sem_s = scr[3 * NIDX + 2 * NROWBUF:3 * NIDX + 3 * NROWBUF]
        sem_r = scr[3 * NIDX + 3 * NROWBUF:]

        c = lax.axis_index("c")
        s = lax.axis_index("s")
        rbase = s * ROWS_PER_TILE
        tid = c * NS + s
        ebase = tid * (CPT * CHUNK)

        def refill(q, j):
            off = ebase + q * CHUNK
            pltpu.make_async_copy(
                col_hbm.at[pl.ds(off, CHUNK)], colr[j], sem_r[j]).start()
            pltpu.make_async_copy(
                row_hbm.at[pl.ds(off, CHUNK)], rowr[j], sem_r[j]).start()
            pltpu.make_async_copy(
                w_hbm.at[pl.ds(off, CHUNK)], wr[j], sem_r[j]).start()

        def wait_idx(q, j):
            off = ebase + q * CHUNK
            pltpu.make_async_copy(
                col_hbm.at[pl.ds(off, CHUNK)], colr[j], sem_r[j]).wait()
            pltpu.make_async_copy(
                row_hbm.at[pl.ds(off, CHUNK)], rowr[j], sem_r[j]).wait()
            pltpu.make_async_copy(
                w_hbm.at[pl.ds(off, CHUNK)], wr[j], sem_r[j]).wait()

        def gather_start(b, j):
            pltpu.make_async_copy(ego_hbm.at[colr[j]], rows[b], sem_g[b]).start()

        def gather_wait(b, j):
            pltpu.make_async_copy(ego_hbm.at[colr[j]], rows[b], sem_g[b]).wait()

        def scatter_start(b, j):
            pltpu.make_async_copy(
                rows[b], agg_sh.at[rowr[j]], sem_s[b]).start(add=True)

        def scatter_wait(b, j):
            pltpu.make_async_copy(
                rows[b], agg_sh.at[rowr[j]], sem_s[b]).wait()

        def mult(b, j):
            def mgrp(kk, inner):
                wvec = wr[j][pl.ds(kk * 16, 16)]
                for jj in range(16):
                    i = kk * 16 + jj
                    lane = jnp.full((16,), jj, dtype=jnp.int32)
                    wsplat = wvec.at[lane].get(mode="promise_in_bounds")
                    for d8 in range(D // 16):
                        sl = pl.ds(d8 * 16, 16)
                        rows[b][i, sl] = rows[b][i, sl] * wsplat
                return inner
            lax.fori_loop(0, CHUNK // 16, mgrp, 0)

        def step(q, pos, do_swait, do_gather, do_refill):
            b = pos % NROWBUF
            j = pos % NIDX
            b2 = (pos + 2) % NROWBUF
            j2 = (pos + 2) % NIDX
            gather_wait(b, j)
            mult(b, j)
            scatter_start(b, j)
            if do_swait:
                scatter_wait(b2, j2)
            if do_gather:
                wait_idx(q + 2, j2)
                gather_start(b2, j2)
            if do_refill:
                refill(q + 4, (pos + 4) % NIDX)

        for q in range(4):
            refill(q, q)

        def zrow(i, carry):
            for d8 in range(D // 16):
                rows[0][i, pl.ds(d8 * 16, 16)] = jnp.zeros((16,), jnp.float32)
            return carry
        lax.fori_loop(0, CHUNK, zrow, 0)
        for kk in range(ROWS_PER_TILE // CHUNK):
            pltpu.sync_copy(rows[0],
                            agg_sh.at[pl.ds(rbase + kk * CHUNK, CHUNK)])
        wait_idx(0, 0)
        wait_idx(1, 1)
        plsc.subcore_barrier()
        gather_start(0, 0)
        gather_start(1, 1)

        for pos in range(8):
            step(pos, pos, do_swait=(pos >= 2), do_gather=True, do_refill=True)

        def body(p, carry):
            q0 = 8 * p
            for pos in range(8):
                step(q0 + pos, pos, True, True, True)
            return carry
        lax.fori_loop(1, CPT // 8 - 1, body, 0)

        q0 = CPT - 8
        for pos in range(8):
            step(q0 + pos, pos,
                 do_swait=True,
                 do_gather=(q0 + pos + 2 <= CPT - 1),
                 do_refill=(q0 + pos + 4 <= CPT - 1))

        scatter_wait((CPT - 2) % NROWBUF, (CPT - 2) % NIDX)
        scatter_wait((CPT - 1) % NROWBUF, (CPT - 1) % NIDX)
        plsc.subcore_barrier()
        for kk in range(ROWS_PER_TILE // CHUNK):
            r0 = rbase + kk * CHUNK
            pltpu.sync_copy(agg_sh.at[pl.ds(r0, CHUNK)], rows[0])
            pltpu.sync_copy(rows[0], out_hbm.at[pl.ds(c * NP_ + r0, CHUNK)])

    return k(ego, col, row, w)


def _tc_update(partials, ego, acc, scale):
    bn = 1000

    def body(p_ref, e_ref, a_ref, eo_ref, ao_ref):
        agg = p_ref[0] + p_ref[1]
        e_new = agg + agg * e_ref[...]
        eo_ref[...] = e_new
        ao_ref[...] = (a_ref[...] + e_new) * scale

    return pl.pallas_call(
        body,
        grid=(N // bn,),
        in_specs=[
            pl.BlockSpec((2, bn, D), lambda i: (0, i, 0)),
            pl.BlockSpec((bn, D), lambda i: (i, 0)),
            pl.BlockSpec((bn, D), lambda i: (i, 0)),
        ],
        out_specs=[
            pl.BlockSpec((bn, D), lambda i: (i, 0)),
            pl.BlockSpec((bn, D), lambda i: (i, 0)),
        ],
        out_shape=[
            jax.ShapeDtypeStruct((N, D), jnp.float32),
            jax.ShapeDtypeStruct((N, D), jnp.float32),
        ],
    )(partials, ego, acc)


def kernel(X, edge_index, edge_weight):
    row = edge_index[0]
    col = edge_index[1]
    e = row.shape[0]
    e_pad = NC * NS * CPT * CHUNK
    pad = e_pad - e
    if pad > 0:
        pad_idx = jnp.arange(pad, dtype=jnp.int32) % N
        row = jnp.concatenate([row, pad_idx])
        col = jnp.concatenate([col, pad_idx])
        w = jnp.concatenate([edge_weight, jnp.zeros((pad,), jnp.float32)])
    else:
        w = edge_weight

    ego = X
    acc = X
    for layer in range(LAYERS):
        partials = _sc_agg(ego, col, row, w).reshape(2, NP_, D)
        scale = (1.0 / (LAYERS + 1)) if layer == LAYERS - 1 else 1.0
        ego, acc = _tc_update(partials, ego, acc, scale)
    return acc


# speedup vs baseline: 1.0670x; 1.0670x over previous
import functools

import jax
import jax.numpy as jnp
from jax import lax
from jax.experimental import pallas as pl
from jax.experimental.pallas import tpu as pltpu
from jax.experimental.pallas import tpu_sc as plsc

N = 10000
D = 128
LAYERS = 2
NC = 2
NS = 16
CHUNK = 80
NP_ = 10240
ROWS_PER_TILE = NP_ // NS
CPT = 128
NROWBUF = 4
NIDX = 8


def _sc_agg(ego, col, row, w):
    mesh = plsc.VectorSubcoreMesh(core_axis_name="c", subcore_axis_name="s")

    @functools.partial(
        pl.kernel,
        mesh=mesh,
        out_type=jax.ShapeDtypeStruct((NC * NP_, D), jnp.float32),
        scratch_types=(
            [pltpu.VMEM_SHARED((NP_, D), jnp.float32)]
            + [pltpu.VMEM((CHUNK,), jnp.int32)] * NIDX
            + [pltpu.VMEM((CHUNK,), jnp.int32)] * NIDX
            + [pltpu.VMEM((CHUNK,), jnp.float32)] * NIDX
            + [pltpu.VMEM((CHUNK, D), jnp.float32)] * NROWBUF
            + [pltpu.SemaphoreType.DMA] * NROWBUF
            + [pltpu.SemaphoreType.DMA] * NROWBUF
            + [pltpu.SemaphoreType.DMA] * NIDX
        ),
    )
    def k(ego_hbm, col_hbm, row_hbm, w_hbm, out_hbm, agg_sh, *scr):
        colr = scr[0:NIDX]
        rowr = scr[NIDX:2 * NIDX]
        wr = scr[2 * NIDX:3 * NIDX]
        rows = scr[3 * NIDX:3 * NIDX + NROWBUF]
        sem_g = scr[3 * NIDX + NROWBUF:3 * NIDX + 2 * NROWBUF]
        sem_s = scr[3 * NIDX + 2 * NROWBUF:3 * NIDX + 3 * NROWBUF]
        sem_r = scr[3 * NIDX + 3 * NROWBUF:]

        c = lax.axis_index("c")
        s = lax.axis_index("s")
        rbase = s * ROWS_PER_TILE
        tid = c * NS + s
        ebase = tid * (CPT * CHUNK)

        def refill(q, j):
            off = ebase + q * CHUNK
            pltpu.make_async_copy(
                col_hbm.at[pl.ds(off, CHUNK)], colr[j], sem_r[j]).start()
            pltpu.make_async_copy(
                row_hbm.at[pl.ds(off, CHUNK)], rowr[j], sem_r[j]).start()
            pltpu.make_async_copy(
                w_hbm.at[pl.ds(off, CHUNK)], wr[j], sem_r[j]).start()

        def wait_idx(q, j):
            off = ebase + q * CHUNK
            pltpu.make_async_copy(
                col_hbm.at[pl.ds(off, CHUNK)], colr[j], sem_r[j]).wait()
            pltpu.make_async_copy(
                row_hbm.at[pl.ds(off, CHUNK)], rowr[j], sem_r[j]).wait()
            pltpu.make_async_copy(
                w_hbm.at[pl.ds(off, CHUNK)], wr[j], sem_r[j]).wait()

        def gather_start(b, j):
            pltpu.make_async_copy(ego_hbm.at[colr[j]], rows[b], sem_g[b]).start()

        def gather_wait(b, j):
            pltpu.make_async_copy(ego_hbm.at[colr[j]], rows[b], sem_g[b]).wait()

        def scatter_start(b, j):
            pltpu.make_async_copy(
                rows[b], agg_sh.at[rowr[j]], sem_s[b]).start(add=True)

        def scatter_wait(b, j):
            pltpu.make_async_copy(
                rows[b], agg_sh.at[rowr[j]], sem_s[b]).wait()

        def mult(b, j):
            def mgrp(kk, inner):
                wvec = wr[j][pl.ds(kk * 16, 16)]
                for jj in range(16):
                    i = kk * 16 + jj
                    lane = jnp.full((16,), jj, dtype=jnp.int32)
                    wsplat = wvec.at[lane].get(mode="promise_in_bounds")
                    for d8 in range(D // 16):
                        sl = pl.ds(d8 * 16, 16)
                        rows[b][i, sl] = rows[b][i, sl] * wsplat
                return inner
            lax.fori_loop(0, CHUNK // 16, mgrp, 0)

        def step(q, pos, do_swait, do_gather, do_refill):
            b = pos % NROWBUF
            j = pos % NIDX
            b2 = (pos + 2) % NROWBUF
            j2 = (pos + 2) % NIDX
            gather_wait(b, j)
            mult(b, j)
            scatter_start(b, j)
            if do_swait:
                scatter_wait(b2, j2)
            if do_gather:
                wait_idx(q + 2, j2)
                gather_start(b2, j2)
            if do_refill:
                refill(q + 4, (pos + 4) % NIDX)

        for q in range(4):
            refill(q, q)

        def zrow(i, carry):
            for d8 in range(D // 16):
                rows[0][i, pl.ds(d8 * 16, 16)] = jnp.zeros((16,), jnp.float32)
            return carry
        lax.fori_loop(0, CHUNK, zrow, 0)
        for kk in range(ROWS_PER_TILE // CHUNK):
            pltpu.make_async_copy(
                rows[0], agg_sh.at[pl.ds(rbase + kk * CHUNK, CHUNK)],
                sem_s[0]).start()
        for kk in range(ROWS_PER_TILE // CHUNK):
            pltpu.make_async_copy(
                rows[0], agg_sh.at[pl.ds(rbase + kk * CHUNK, CHUNK)],
                sem_s[0]).wait()
        wait_idx(0, 0)
        wait_idx(1, 1)
        plsc.subcore_barrier()
        gather_start(0, 0)
        gather_start(1, 1)

        for pos in range(8):
            step(pos, pos, do_swait=(pos >= 2), do_gather=True, do_refill=True)

        def body(p, carry):
            q0 = 8 * p
            for pos in range(8):
                step(q0 + pos, pos, True, True, True)
            return carry
        lax.fori_loop(1, CPT // 8 - 1, body, 0)

        q0 = CPT - 8
        for pos in range(8):
            step(q0 + pos, pos,
                 do_swait=True,
                 do_gather=(q0 + pos + 2 <= CPT - 1),
                 do_refill=(q0 + pos + 4 <= CPT - 1))

        scatter_wait((CPT - 2) % NROWBUF, (CPT - 2) % NIDX)
        scatter_wait((CPT - 1) % NROWBUF, (CPT - 1) % NIDX)
        plsc.subcore_barrier()
        nw = ROWS_PER_TILE // CHUNK
        for kk in range(nw):
            b = kk % 2
            r0 = rbase + kk * CHUNK
            if kk >= 2:
                rp = rbase + (kk - 2) * CHUNK
                pltpu.make_async_copy(
                    rows[b], out_hbm.at[pl.ds(c * NP_ + rp, CHUNK)],
                    sem_g[b]).wait()
            pltpu.sync_copy(agg_sh.at[pl.ds(r0, CHUNK)], rows[b])
            pltpu.make_async_copy(
                rows[b], out_hbm.at[pl.ds(c * NP_ + r0, CHUNK)],
                sem_g[b]).start()
        for kk in range(nw - 2, nw):
            b = kk % 2
            rp = rbase + kk * CHUNK
            pltpu.make_async_copy(
                rows[b], out_hbm.at[pl.ds(c * NP_ + rp, CHUNK)],
                sem_g[b]).wait()

    return k(ego, col, row, w)


def _tc_update(partials, ego, acc, scale):
    bn = 1000

    def body(p_ref, e_ref, a_ref, eo_ref, ao_ref):
        agg = p_ref[0] + p_ref[1]
        e_new = agg + agg * e_ref[...]
        eo_ref[...] = e_new
        ao_ref[...] = (a_ref[...] + e_new) * scale

    return pl.pallas_call(
        body,
        grid=(N // bn,),
        in_specs=[
            pl.BlockSpec((2, bn, D), lambda i: (0, i, 0)),
            pl.BlockSpec((bn, D), lambda i: (i, 0)),
            pl.BlockSpec((bn, D), lambda i: (i, 0)),
        ],
        out_specs=[
            pl.BlockSpec((bn, D), lambda i: (i, 0)),
            pl.BlockSpec((bn, D), lambda i: (i, 0)),
        ],
        out_shape=[
            jax.ShapeDtypeStruct((N, D), jnp.float32),
            jax.ShapeDtypeStruct((N, D), jnp.float32),
        ],
    )(partials, ego, acc)


def kernel(X, edge_index, edge_weight):
    row = edge_index[0]
    col = edge_index[1]
    e = row.shape[0]
    e_pad = NC * NS * CPT * CHUNK
    pad = e_pad - e
    if pad > 0:
        pad_idx = jnp.arange(pad, dtype=jnp.int32) % N
        row = jnp.concatenate([row, pad_idx])
        col = jnp.concatenate([col, pad_idx])
        w = jnp.concatenate([edge_weight, jnp.zeros((pad,), jnp.float32)])
    else:
        w = edge_weight

    ego = X
    acc = X
    for layer in range(LAYERS):
        partials = _sc_agg(ego, col, row, w).reshape(2, NP_, D)
        scale = (1.0 / (LAYERS + 1)) if layer == LAYERS - 1 else 1.0
        ego, acc = _tc_update(partials, ego, acc, scale)
    return acc
